# baseline (speedup 1.0000x reference)
import jax
import jax.numpy as jnp
from jax.experimental import pallas as pl
from jax.experimental.pallas import tpu as pltpu

B, N, D, H, T = 8, 1024, 128, 32, 3


def _dot(a, b):
    return jax.lax.dot_general(
        a, b,
        (((a.ndim - 1,), (0,)), ((), ())),
        preferred_element_type=jnp.float32)


def _gru(a, x, wr, wz, wn, ur, uz, un, br, bz, bn, cr, cz, cn):
    r = jax.nn.sigmoid(_dot(a, wr) + br + _dot(x, ur) + cr)
    z = jax.nn.sigmoid(_dot(a, wz) + bz + _dot(x, uz) + cz)
    n = jnp.tanh(_dot(a, wn) + bn + r * (_dot(x, un) + cn))
    return (1.0 - z) * n + z * x


def _body(x_padded_ref, edges_ref, fc_wT_ref, fc_b_ref,
          W1_ref, g1_refs, W2_ref, g2_refs, W3_ref, g3_refs,
          out_wT_ref, out_b_ref, out_ref, x_s, a_s):
    x_s[...] = _dot(x_padded_ref[0], fc_wT_ref[:]) + fc_b_ref[:]

    for W_ref, g_refs in ((W1_ref, g1_refs), (W2_ref, g2_refs)):
        x = x_s[...]
        a = _dot(edges_ref[0, 0], _dot(x, W_ref[0]))
        a += _dot(edges_ref[0, 1], _dot(x, W_ref[1]))
        a += _dot(edges_ref[0, 2], _dot(x, W_ref[2]))
        a_s[...] = a
        x_s[...] = _gru(a_s[...], x, *[g[:] for g in g_refs])

    x = x_s[...]
    a3 = _dot(edges_ref[0, 0, 5:6, :], _dot(x, W3_ref[0]))
    a3 += _dot(edges_ref[0, 1, 5:6, :], _dot(x, W3_ref[1]))
    a3 += _dot(edges_ref[0, 2, 5:6, :], _dot(x, W3_ref[2]))
    h = _gru(a3, x_s[5:6, :], *[g[:] for g in g3_refs])

    logits = _dot(h, out_wT_ref[:]) + out_b_ref[:]
    mx = jnp.max(logits, axis=1, keepdims=True)
    lse = mx + jnp.log(jnp.sum(jnp.exp(logits - mx), axis=1, keepdims=True))
    out_ref[0] = logits - lse


@jax.jit
def kernel(x_padded, x_lengths, edges, fc_w, fc_b,
           W1, wih1, whh1, bih1, bhh1,
           W2, wih2, whh2, bih2, bhh2,
           W3, wih3, whh3, bih3, bhh3,
           out_w, out_b):
    del x_lengths

    def gru_params(wih, whh, bih, bhh):
        return (tuple(wih[k * H:(k + 1) * H].T for k in range(3))
                + tuple(whh[k * H:(k + 1) * H].T for k in range(3))
                + tuple(bih[k * H:(k + 1) * H].reshape(1, H) for k in range(3))
                + tuple(bhh[k * H:(k + 1) * H].reshape(1, H) for k in range(3)))

    row2 = lambda v: v.reshape(1, -1)
    ins = (
        x_padded, edges,
        fc_w.T, row2(fc_b),
        W1, gru_params(wih1, whh1, bih1, bhh1),
        W2, gru_params(wih2, whh2, bih2, bhh2),
        W3, gru_params(wih3, whh3, bih3, bhh3),
        out_w.T, row2(out_b),
    )
    flat, treedef = jax.tree.flatten(ins)

    def full(x):
        return pl.BlockSpec(x.shape, lambda b: (0,) * x.ndim)

    specs = [
        pl.BlockSpec((1, N, D), lambda b: (b, 0, 0)),
        pl.BlockSpec((1, T, N, N), lambda b: (b, 0, 0, 0)),
    ] + [full(x) for x in flat[2:]]
    specs = jax.tree.unflatten(treedef, specs)

    out = pl.pallas_call(
        _body,
        grid=(B,),
        in_specs=specs,
        out_specs=pl.BlockSpec((1, 1, 5), lambda b: (b, 0, 0)),
        out_shape=jax.ShapeDtypeStruct((B, 1, 5), jnp.float32),
        scratch_shapes=[pltpu.VMEM((N, H), jnp.float32),
                        pltpu.VMEM((N, H), jnp.float32)],
        compiler_params=pltpu.CompilerParams(
            dimension_semantics=("arbitrary",)),
    )(*ins)
    return out.reshape(B, 5)


# speedup vs baseline: 1.4424x; 1.4424x over previous
import jax
import jax.numpy as jnp
from jax.experimental import pallas as pl
from jax.experimental.pallas import tpu as pltpu

B, N, D, H, T = 8, 1024, 128, 32, 3
NSHARDS = 4
RS = N // NSHARDS


def _dot(a, b):
    return jax.lax.dot_general(
        a, b,
        (((a.ndim - 1,), (0,)), ((), ())),
        preferred_element_type=jnp.float32)


def _gru(a, x, wr, wz, wn, ur, uz, un, br, bz, bn, cr, cz, cn):
    r = jax.nn.sigmoid(_dot(a, wr) + br + _dot(x, ur) + cr)
    z = jax.nn.sigmoid(_dot(a, wz) + bz + _dot(x, uz) + cz)
    n = jnp.tanh(_dot(a, wn) + bn + r * (_dot(x, un) + cn))
    return (1.0 - z) * n + z * x


def _body(x_padded_ref, e0_ref, e1_ref, e2_ref, e3_ref, fc_wT_ref, fc_b_ref,
          W1_ref, g1_refs, W2_ref, g2_refs, W3_ref, g3_refs,
          out_wT_ref, out_b_ref, out_ref, x_s, a_s):
    e_refs = (e0_ref, e1_ref, e2_ref, e3_ref)
    x_s[...] = _dot(x_padded_ref[0], fc_wT_ref[:]) + fc_b_ref[:]

    for W_ref, g_refs in ((W1_ref, g1_refs), (W2_ref, g2_refs)):
        x = x_s[...]
        m0 = _dot(x, W_ref[0])
        m1 = _dot(x, W_ref[1])
        m2 = _dot(x, W_ref[2])
        for i, e_ref in enumerate(e_refs):
            ai = _dot(e_ref[0, 0], m0)
            ai += _dot(e_ref[0, 1], m1)
            ai += _dot(e_ref[0, 2], m2)
            a_s[i * RS:(i + 1) * RS, :] = ai
        x_s[...] = _gru(a_s[...], x, *[g[:] for g in g_refs])

    x = x_s[...]
    a3 = _dot(e0_ref[0, 0, 5:6, :], _dot(x, W3_ref[0]))
    a3 += _dot(e0_ref[0, 1, 5:6, :], _dot(x, W3_ref[1]))
    a3 += _dot(e0_ref[0, 2, 5:6, :], _dot(x, W3_ref[2]))
    h = _gru(a3, x_s[5:6, :], *[g[:] for g in g3_refs])

    logits = _dot(h, out_wT_ref[:]) + out_b_ref[:]
    mx = jnp.max(logits, axis=1, keepdims=True)
    lse = mx + jnp.log(jnp.sum(jnp.exp(logits - mx), axis=1, keepdims=True))
    out_ref[0] = logits - lse


def _shard_spec(i):
    return pl.BlockSpec((1, T, RS, N), lambda b, i=i: (b, 0, i, 0))


@jax.jit
def kernel(x_padded, x_lengths, edges, fc_w, fc_b,
           W1, wih1, whh1, bih1, bhh1,
           W2, wih2, whh2, bih2, bhh2,
           W3, wih3, whh3, bih3, bhh3,
           out_w, out_b):
    del x_lengths

    def gru_params(wih, whh, bih, bhh):
        return (tuple(wih[k * H:(k + 1) * H].T for k in range(3))
                + tuple(whh[k * H:(k + 1) * H].T for k in range(3))
                + tuple(bih[k * H:(k + 1) * H].reshape(1, H) for k in range(3))
                + tuple(bhh[k * H:(k + 1) * H].reshape(1, H) for k in range(3)))

    row2 = lambda v: v.reshape(1, -1)
    ins = (
        x_padded, edges, edges, edges, edges,
        fc_w.T, row2(fc_b),
        W1, gru_params(wih1, whh1, bih1, bhh1),
        W2, gru_params(wih2, whh2, bih2, bhh2),
        W3, gru_params(wih3, whh3, bih3, bhh3),
        out_w.T, row2(out_b),
    )
    flat, treedef = jax.tree.flatten(ins)

    def full(x):
        return pl.BlockSpec(x.shape, lambda b: (0,) * x.ndim)

    specs = [
        pl.BlockSpec((1, N, D), lambda b: (b, 0, 0)),
    ] + [_shard_spec(i) for i in range(NSHARDS)] + [full(x) for x in flat[5:]]
    specs = jax.tree.unflatten(treedef, specs)

    out = pl.pallas_call(
        _body,
        grid=(B,),
        in_specs=specs,
        out_specs=pl.BlockSpec((1, 1, 5), lambda b: (b, 0, 0)),
        out_shape=jax.ShapeDtypeStruct((B, 1, 5), jnp.float32),
        scratch_shapes=[pltpu.VMEM((N, H), jnp.float32),
                        pltpu.VMEM((N, H), jnp.float32)],
        compiler_params=pltpu.CompilerParams(
            dimension_semantics=("arbitrary",)),
    )(*ins)
    return out.reshape(B, 5)


# speedup vs baseline: 1.6653x; 1.1545x over previous
import jax
import jax.numpy as jnp
from jax.experimental import pallas as pl
from jax.experimental.pallas import tpu as pltpu

B, N, D, H, T = 8, 1024, 128, 32, 3
PB = 2


def _dot(a, b):
    return jax.lax.dot_general(
        a, b,
        (((a.ndim - 1,), (0,)), ((), ())),
        preferred_element_type=jnp.float32)


def _gru(a, x, wihT, bih, whhT, bhh):
    gi = _dot(a, wihT) + bih
    gh = _dot(x, whhT) + bhh
    r = jax.nn.sigmoid(gi[:, :H] + gh[:, :H])
    z = jax.nn.sigmoid(gi[:, H:2 * H] + gh[:, H:2 * H])
    n = jnp.tanh(gi[:, 2 * H:] + r * gh[:, 2 * H:])
    return (1.0 - z) * n + z * x


def _body(x_padded_ref, edges_ref, fc_wT_ref, fc_b_ref,
          W1_ref, wih1T_ref, whh1T_ref, bih1_ref, bhh1_ref,
          W2_ref, wih2T_ref, whh2T_ref, bih2_ref, bhh2_ref,
          W3_ref, wih3T_ref, whh3T_ref, bih3_ref, bhh3_ref,
          out_wT_ref, out_b_ref, out_ref, x_s, a_s):
    for bb in range(PB):
        x_s[bb] = _dot(x_padded_ref[bb], fc_wT_ref[:]) + fc_b_ref[:]

    for (W_ref, wihT_ref, whhT_ref, bih_ref, bhh_ref) in (
            (W1_ref, wih1T_ref, whh1T_ref, bih1_ref, bhh1_ref),
            (W2_ref, wih2T_ref, whh2T_ref, bih2_ref, bhh2_ref)):
        for bb in range(PB):
            x = x_s[bb]
            ai = _dot(edges_ref[bb, 0], _dot(x, W_ref[0]))
            ai += _dot(edges_ref[bb, 1], _dot(x, W_ref[1]))
            ai += _dot(edges_ref[bb, 2], _dot(x, W_ref[2]))
            a_s[bb] = ai
        for bb in range(PB):
            x_s[bb] = _gru(a_s[bb], x_s[bb], wihT_ref[:], bih_ref[:],
                           whhT_ref[:], bhh_ref[:])

    for bb in range(PB):
        x = x_s[bb]
        a3 = _dot(edges_ref[bb, 0, 5:6, :], _dot(x, W3_ref[0]))
        a3 += _dot(edges_ref[bb, 1, 5:6, :], _dot(x, W3_ref[1]))
        a3 += _dot(edges_ref[bb, 2, 5:6, :], _dot(x, W3_ref[2]))
        h = _gru(a3, x_s[bb, 5:6, :], wih3T_ref[:], bih3_ref[:],
                 whh3T_ref[:], bhh3_ref[:])

        logits = _dot(h, out_wT_ref[:]) + out_b_ref[:]
        mx = jnp.max(logits, axis=1, keepdims=True)
        lse = mx + jnp.log(jnp.sum(jnp.exp(logits - mx), axis=1,
                                   keepdims=True))
        out_ref[bb] = logits - lse


@jax.jit
def kernel(x_padded, x_lengths, edges, fc_w, fc_b,
           W1, wih1, whh1, bih1, bhh1,
           W2, wih2, whh2, bih2, bhh2,
           W3, wih3, whh3, bih3, bhh3,
           out_w, out_b):
    del x_lengths

    def full(x):
        return pl.BlockSpec(x.shape, lambda b: (0,) * x.ndim)

    row2 = lambda v: v.reshape(1, -1)
    ins = (
        x_padded, edges,
        fc_w.T, row2(fc_b),
        W1, wih1.T, whh1.T, row2(bih1), row2(bhh1),
        W2, wih2.T, whh2.T, row2(bih2), row2(bhh2),
        W3, wih3.T, whh3.T, row2(bih3), row2(bhh3),
        out_w.T, row2(out_b),
    )
    specs = [
        pl.BlockSpec((PB, N, D), lambda b: (b, 0, 0)),
        pl.BlockSpec((PB, T, N, N), lambda b: (b, 0, 0, 0)),
    ] + [full(x) for x in ins[2:]]

    out = pl.pallas_call(
        _body,
        grid=(B // PB,),
        in_specs=specs,
        out_specs=pl.BlockSpec((PB, 1, 5), lambda b: (b, 0, 0)),
        out_shape=jax.ShapeDtypeStruct((B, 1, 5), jnp.float32),
        scratch_shapes=[pltpu.VMEM((PB, N, H), jnp.float32),
                        pltpu.VMEM((PB, N, H), jnp.float32)],
        compiler_params=pltpu.CompilerParams(
            dimension_semantics=("arbitrary",)),
    )(*ins)
    return out.reshape(B, 5)


# speedup vs baseline: 1.7309x; 1.0394x over previous
import jax
import jax.numpy as jnp
from jax.experimental import pallas as pl
from jax.experimental.pallas import tpu as pltpu

B, N, D, H, T = 8, 1024, 128, 32, 3
PB = 2


def _dot(a, b):
    return jax.lax.dot_general(
        a, b,
        (((a.ndim - 1,), (0,)), ((), ())),
        preferred_element_type=jnp.float32)


def _gru(a, x, wihT, bih, whhT, bhh):
    gi = _dot(a, wihT) + bih
    gh = _dot(x, whhT) + bhh
    r = jax.nn.sigmoid(gi[:, :H] + gh[:, :H])
    z = jax.nn.sigmoid(gi[:, H:2 * H] + gh[:, H:2 * H])
    n = jnp.tanh(gi[:, 2 * H:] + r * gh[:, 2 * H:])
    return (1.0 - z) * n + z * x


def _body(x_padded_ref, edges_ref, fc_wT_ref, fc_b_ref,
          W1_ref, wih1T_ref, whh1T_ref, bih1_ref, bhh1_ref,
          W2_ref, wih2T_ref, whh2T_ref, bih2_ref, bhh2_ref,
          W3_ref, wih3T_ref, whh3T_ref, bih3_ref, bhh3_ref,
          out_wT_ref, out_b_ref, out_ref, x_s, a_s):
    for bb in range(PB):
        x_s[bb] = _dot(x_padded_ref[bb], fc_wT_ref[:]) + fc_b_ref[:]

    for (W_ref, wihT_ref, whhT_ref, bih_ref, bhh_ref) in (
            (W1_ref, wih1T_ref, whh1T_ref, bih1_ref, bhh1_ref),
            (W2_ref, wih2T_ref, whh2T_ref, bih2_ref, bhh2_ref)):
        for bb in range(PB):
            x = x_s[bb]
            ai = _dot(edges_ref[bb, 0], _dot(x, W_ref[0]))
            ai += _dot(edges_ref[bb, 1], _dot(x, W_ref[1]))
            ai += _dot(edges_ref[bb, 2], _dot(x, W_ref[2]))
            a_s[bb] = ai
        for bb in range(PB):
            for blk in range(4):
                rows = slice(blk * (N // 4), (blk + 1) * (N // 4))
                x_s[bb, rows] = _gru(a_s[bb, rows], x_s[bb, rows],
                                     wihT_ref[:], bih_ref[:],
                                     whhT_ref[:], bhh_ref[:])

    for bb in range(PB):
        x = x_s[bb]
        a3 = _dot(edges_ref[bb, 0, 5:6, :], _dot(x, W3_ref[0]))
        a3 += _dot(edges_ref[bb, 1, 5:6, :], _dot(x, W3_ref[1]))
        a3 += _dot(edges_ref[bb, 2, 5:6, :], _dot(x, W3_ref[2]))
        h = _gru(a3, x_s[bb, 5:6, :], wih3T_ref[:], bih3_ref[:],
                 whh3T_ref[:], bhh3_ref[:])

        logits = _dot(h, out_wT_ref[:]) + out_b_ref[:]
        mx = jnp.max(logits, axis=1, keepdims=True)
        lse = mx + jnp.log(jnp.sum(jnp.exp(logits - mx), axis=1,
                                   keepdims=True))
        out_ref[bb] = logits - lse


@jax.jit
def kernel(x_padded, x_lengths, edges, fc_w, fc_b,
           W1, wih1, whh1, bih1, bhh1,
           W2, wih2, whh2, bih2, bhh2,
           W3, wih3, whh3, bih3, bhh3,
           out_w, out_b):
    del x_lengths

    def full(x):
        return pl.BlockSpec(x.shape, lambda b: (0,) * x.ndim)

    row2 = lambda v: v.reshape(1, -1)
    ins = (
        x_padded, edges,
        fc_w.T, row2(fc_b),
        W1, wih1.T, whh1.T, row2(bih1), row2(bhh1),
        W2, wih2.T, whh2.T, row2(bih2), row2(bhh2),
        W3, wih3.T, whh3.T, row2(bih3), row2(bhh3),
        out_w.T, row2(out_b),
    )
    specs = [
        pl.BlockSpec((PB, N, D), lambda b: (b, 0, 0)),
        pl.BlockSpec((PB, T, N, N), lambda b: (b, 0, 0, 0)),
    ] + [full(x) for x in ins[2:]]

    out = pl.pallas_call(
        _body,
        grid=(B // PB,),
        in_specs=specs,
        out_specs=pl.BlockSpec((PB, 1, 5), lambda b: (b, 0, 0)),
        out_shape=jax.ShapeDtypeStruct((B, 1, 5), jnp.float32),
        scratch_shapes=[pltpu.VMEM((PB, N, H), jnp.float32),
                        pltpu.VMEM((PB, N, H), jnp.float32)],
        compiler_params=pltpu.CompilerParams(
            dimension_semantics=("arbitrary",)),
    )(*ins)
    return out.reshape(B, 5)


# speedup vs baseline: 1.7309x; 1.0000x over previous
import jax
import jax.numpy as jnp
from jax.experimental import pallas as pl
from jax.experimental.pallas import tpu as pltpu

B, N, D, H, T = 8, 1024, 128, 32, 3
PB = 2


def _dot(a, b):
    return jax.lax.dot_general(
        a, b,
        (((a.ndim - 1,), (0,)), ((), ())),
        preferred_element_type=jnp.float32)


def _sigmoid(v):
    return 0.5 * jnp.tanh(0.5 * v) + 0.5


def _gru(a, x, wihT, bih, whhT, bhh):
    gi = _dot(a, wihT) + bih
    gh = _dot(x, whhT) + bhh
    r = _sigmoid(gi[:, :H] + gh[:, :H])
    z = _sigmoid(gi[:, H:2 * H] + gh[:, H:2 * H])
    n = jnp.tanh(gi[:, 2 * H:] + r * gh[:, 2 * H:])
    return (1.0 - z) * n + z * x


def _body(x_padded_ref, edges_ref, fc_wT_ref, fc_b_ref,
          W1_ref, wih1T_ref, whh1T_ref, bih1_ref, bhh1_ref,
          W2_ref, wih2T_ref, whh2T_ref, bih2_ref, bhh2_ref,
          W3_ref, wih3T_ref, whh3T_ref, bih3_ref, bhh3_ref,
          out_wT_ref, out_b_ref, out_ref, x_s, a_s):
    for bb in range(PB):
        x_s[bb] = _dot(x_padded_ref[bb], fc_wT_ref[:]) + fc_b_ref[:]

    for (W_ref, wihT_ref, whhT_ref, bih_ref, bhh_ref) in (
            (W1_ref, wih1T_ref, whh1T_ref, bih1_ref, bhh1_ref),
            (W2_ref, wih2T_ref, whh2T_ref, bih2_ref, bhh2_ref)):
        for bb in range(PB):
            x = x_s[bb]
            ai = _dot(edges_ref[bb, 0], _dot(x, W_ref[0]))
            ai += _dot(edges_ref[bb, 1], _dot(x, W_ref[1]))
            ai += _dot(edges_ref[bb, 2], _dot(x, W_ref[2]))
            a_s[bb] = ai
        for bb in range(PB):
            for blk in range(4):
                rows = slice(blk * (N // 4), (blk + 1) * (N // 4))
                x_s[bb, rows] = _gru(a_s[bb, rows], x_s[bb, rows],
                                     wihT_ref[:], bih_ref[:],
                                     whhT_ref[:], bhh_ref[:])

    for bb in range(PB):
        x = x_s[bb]
        a3 = _dot(edges_ref[bb, 0, 5:6, :], _dot(x, W3_ref[0]))
        a3 += _dot(edges_ref[bb, 1, 5:6, :], _dot(x, W3_ref[1]))
        a3 += _dot(edges_ref[bb, 2, 5:6, :], _dot(x, W3_ref[2]))
        h = _gru(a3, x_s[bb, 5:6, :], wih3T_ref[:], bih3_ref[:],
                 whh3T_ref[:], bhh3_ref[:])

        logits = _dot(h, out_wT_ref[:]) + out_b_ref[:]
        mx = jnp.max(logits, axis=1, keepdims=True)
        lse = mx + jnp.log(jnp.sum(jnp.exp(logits - mx), axis=1,
                                   keepdims=True))
        out_ref[bb] = logits - lse


@jax.jit
def kernel(x_padded, x_lengths, edges, fc_w, fc_b,
           W1, wih1, whh1, bih1, bhh1,
           W2, wih2, whh2, bih2, bhh2,
           W3, wih3, whh3, bih3, bhh3,
           out_w, out_b):
    del x_lengths

    def full(x):
        return pl.BlockSpec(x.shape, lambda b: (0,) * x.ndim)

    row2 = lambda v: v.reshape(1, -1)
    ins = (
        x_padded, edges,
        fc_w.T, row2(fc_b),
        W1, wih1.T, whh1.T, row2(bih1), row2(bhh1),
        W2, wih2.T, whh2.T, row2(bih2), row2(bhh2),
        W3, wih3.T, whh3.T, row2(bih3), row2(bhh3),
        out_w.T, row2(out_b),
    )
    specs = [
        pl.BlockSpec((PB, N, D), lambda b: (b, 0, 0)),
        pl.BlockSpec((PB, T, N, N), lambda b: (b, 0, 0, 0)),
    ] + [full(x) for x in ins[2:]]

    out = pl.pallas_call(
        _body,
        grid=(B // PB,),
        in_specs=specs,
        out_specs=pl.BlockSpec((PB, 1, 5), lambda b: (b, 0, 0)),
        out_shape=jax.ShapeDtypeStruct((B, 1, 5), jnp.float32),
        scratch_shapes=[pltpu.VMEM((PB, N, H), jnp.float32),
                        pltpu.VMEM((PB, N, H), jnp.float32)],
        compiler_params=pltpu.CompilerParams(
            dimension_semantics=("arbitrary",)),
    )(*ins)
    return out.reshape(B, 5)
